# baseline (speedup 1.0000x reference)
import functools

import jax
import jax.numpy as jnp
from jax import lax
from jax.experimental import pallas as pl
from jax.experimental.pallas import tpu as pltpu
from jax.experimental.pallas import tpu_sc as plsc

EPS = 1e-5


def _sc_gather(table, action):
    B = action.shape[0]
    V, D = table.shape
    info = plsc.get_sparse_core_info()
    nc, ns = info.num_cores, info.num_subcores
    nw = nc * ns
    b_per_w = B // nw

    mesh = plsc.VectorSubcoreMesh(core_axis_name="c", subcore_axis_name="s")

    @functools.partial(
        pl.kernel,
        mesh=mesh,
        out_type=jax.ShapeDtypeStruct((B, D), jnp.float32),
        scratch_types=[
            pltpu.VMEM((b_per_w,), jnp.int32),
            pltpu.VMEM((b_per_w, D), jnp.float32),
            pltpu.SemaphoreType.DMA,
        ],
        compiler_params=pltpu.CompilerParams(use_tc_tiling_on_sc=False),
    )
    def gather_kernel(table_hbm, idx_hbm, out_hbm, idx_v, rows_v, sem):
        wid = lax.axis_index("s") * nc + lax.axis_index("c")
        base = wid * b_per_w
        pltpu.sync_copy(idx_hbm.at[pl.ds(base, b_per_w)], idx_v)
        pltpu.async_copy(table_hbm.at[idx_v], rows_v, sem).wait()
        pltpu.sync_copy(rows_v, out_hbm.at[pl.ds(base, b_per_w)])

    return gather_kernel(table, action)


def _bn_tanh(emb, gamma, beta):
    B, D = emb.shape

    def body(e_ref, g_ref, b_ref, o_ref):
        x = e_ref[...]
        mean = jnp.mean(x, axis=0, keepdims=True)
        centered = x - mean
        var = jnp.mean(centered * centered, axis=0, keepdims=True)
        scale = lax.rsqrt(var + EPS) * g_ref[...]
        o_ref[...] = jnp.tanh(centered * scale + b_ref[...])

    return pl.pallas_call(
        body,
        out_shape=jax.ShapeDtypeStruct((B, D), jnp.float32),
    )(emb, gamma.reshape(1, D), beta.reshape(1, D))


@jax.jit
def kernel(action, table, gamma, beta):
    emb = _sc_gather(table, action)
    return _bn_tanh(emb, gamma, beta)


# speedup vs baseline: 1.0157x; 1.0157x over previous
import functools

import jax
import jax.numpy as jnp
from jax import lax
from jax.experimental import pallas as pl
from jax.experimental.pallas import tpu as pltpu
from jax.experimental.pallas import tpu_sc as plsc

EPS = 1e-5


def _sc_gather(table, action):
    V, D = table.shape
    B = action.shape[0]
    info = plsc.get_sparse_core_info()
    nc, ns = info.num_cores, info.num_subcores
    nw = nc * ns
    b_per_w = B // nw

    mesh = plsc.VectorSubcoreMesh(core_axis_name="c", subcore_axis_name="s")

    @functools.partial(
        pl.kernel,
        mesh=mesh,
        compiler_params=pltpu.CompilerParams(use_tc_tiling_on_sc=False),
        out_type=jax.ShapeDtypeStruct((B, D), jnp.float32),
        scratch_types=[
            pltpu.VMEM((b_per_w,), jnp.int32),
            pltpu.VMEM((b_per_w, D), jnp.float32),
            pltpu.SemaphoreType.DMA,
        ],
    )
    def gather_kernel(table_hbm, idx_hbm, out_hbm, idx_v, rows_v, sem):
        wid = lax.axis_index("s") * nc + lax.axis_index("c")
        base = wid * b_per_w
        pltpu.sync_copy(idx_hbm.at[pl.ds(base, b_per_w)], idx_v)
        pltpu.async_copy(table_hbm.at[idx_v], rows_v, sem).wait()
        pltpu.sync_copy(rows_v, out_hbm.at[pl.ds(base, b_per_w)])

    return gather_kernel(table, action)


def _bn_tanh(x, g128, b128, n_rows, n_feat):
    rows, lanes = x.shape
    groups = lanes // n_feat
    inv_n = 1.0 / n_rows

    def body(x_ref, g_ref, b_ref, o_ref):
        v = x_ref[...]
        s = jnp.sum(v, axis=0, keepdims=True)
        ss = jnp.sum(v * v, axis=0, keepdims=True)
        li = lax.broadcasted_iota(jnp.int32, (lanes, lanes), 0) % n_feat
        lj = lax.broadcasted_iota(jnp.int32, (lanes, lanes), 1) % n_feat
        fold = (li == lj).astype(jnp.float32)
        mean = jnp.dot(s, fold, preferred_element_type=jnp.float32) * inv_n
        ex2 = jnp.dot(ss, fold, preferred_element_type=jnp.float32) * inv_n
        var = ex2 - mean * mean
        scale = g_ref[...] * lax.rsqrt(var + EPS)
        o_ref[...] = jnp.tanh((v - mean) * scale + b_ref[...])

    del groups
    return pl.pallas_call(
        body,
        out_shape=jax.ShapeDtypeStruct((rows, lanes), jnp.float32),
    )(x, g128, b128)


@jax.jit
def kernel(action, table, gamma, beta):
    V, D = table.shape
    B = action.shape[0]
    emb = _sc_gather(table, action)
    lanes = 128
    groups = lanes // D
    x = emb.reshape(B * D // lanes, lanes)
    g = jnp.tile(gamma, groups).reshape(1, lanes)
    b = jnp.tile(beta, groups).reshape(1, lanes)
    out = _bn_tanh(x, g, b, B, D)
    return out.reshape(B, D)


# speedup vs baseline: 1.0170x; 1.0013x over previous
import functools

import jax
import jax.numpy as jnp
from jax import lax
from jax.experimental import pallas as pl
from jax.experimental.pallas import tpu as pltpu
from jax.experimental.pallas import tpu_sc as plsc

EPS = 1e-5


def _sc_gather(table, action):
    V, D = table.shape
    B = action.shape[0]
    info = plsc.get_sparse_core_info()
    nw = info.num_cores * info.num_subcores
    b_per_w = B // nw

    mesh = plsc.VectorSubcoreMesh(core_axis_name="c", subcore_axis_name="s")

    @functools.partial(
        pl.kernel,
        mesh=mesh,
        compiler_params=pltpu.CompilerParams(use_tc_tiling_on_sc=False),
        out_type=jax.ShapeDtypeStruct((B, D), jnp.float32),
        scratch_types=[
            pltpu.VMEM((b_per_w,), jnp.int32),
            pltpu.VMEM((b_per_w, D), jnp.float32),
            pltpu.SemaphoreType.DMA,
        ],
    )
    def gather_kernel(table_hbm, idx_hbm, out_hbm, idx_v, rows_v, sem):
        wid = lax.axis_index("s") * info.num_cores + lax.axis_index("c")
        base = wid * b_per_w
        pltpu.sync_copy(idx_hbm.at[pl.ds(base, b_per_w)], idx_v)
        pltpu.async_copy(table_hbm.at[idx_v], rows_v, sem).wait()
        pltpu.sync_copy(rows_v, out_hbm.at[pl.ds(base, b_per_w)])

    return gather_kernel(table, action)


def _bn_tanh(x, g128, b128, n_rows, n_feat):
    rows, lanes = x.shape
    inv_n = 1.0 / n_rows

    def body(x_ref, g_ref, b_ref, o_ref):
        v = x_ref[...]
        s = jnp.sum(v, axis=0, keepdims=True)
        ss = jnp.sum(v * v, axis=0, keepdims=True)
        li = lax.broadcasted_iota(jnp.int32, (lanes, lanes), 0) % n_feat
        lj = lax.broadcasted_iota(jnp.int32, (lanes, lanes), 1) % n_feat
        fold = (li == lj).astype(jnp.float32)
        mean = jnp.dot(s, fold, preferred_element_type=jnp.float32) * inv_n
        ex2 = jnp.dot(ss, fold, preferred_element_type=jnp.float32) * inv_n
        var = ex2 - mean * mean
        scale = g_ref[...] * lax.rsqrt(var + EPS)
        o_ref[...] = jnp.tanh((v - mean) * scale + b_ref[...])

    return pl.pallas_call(
        body,
        out_shape=jax.ShapeDtypeStruct((rows, lanes), jnp.float32),
    )(x, g128, b128)


@jax.jit
def kernel(action, table, gamma, beta):
    V, D = table.shape
    B = action.shape[0]
    lanes = 128
    groups = lanes // D
    gathered = _sc_gather(table, action)
    x = gathered.reshape(B * D // lanes, lanes)
    g = jnp.tile(gamma, groups).reshape(1, lanes)
    b = jnp.tile(beta, groups).reshape(1, lanes)
    out = _bn_tanh(x, g, b, B, D)
    return out.reshape(B, D)
